# Initial kernel scaffold; baseline (speedup 1.0000x reference)
import jax
import jax.numpy as jnp
from jax.experimental import pallas as pl


def kernel(x, lin_tables, ffm_tables, compose_w, compose_b, exc_w1, exc_b1, exc_w2, exc_b2, mlp_w1, mlp_b1, mlp_w2, mlp_b2, mlp_w3, mlp_b3, b_global):
    raise NotImplementedError("write your pallas kernel here")


# speedup vs baseline: 103.3652x; 103.3652x over previous
import functools

import numpy as np
import jax
import jax.numpy as jnp
from jax import lax
from jax.experimental import pallas as pl
from jax.experimental.pallas import tpu as pltpu
from jax.experimental.pallas import tpu_sc as plsc

B = 4096
F = 26
V = 1000
D = 16
C = F * (F - 1) // 2
RED = C // 2
H1, H2 = 1024, 512

_IU, _JU = np.triu_indices(F, k=1)

NC, NS = 2, 16
NW = NC * NS
IDXROW = 128
CG = 8
ROWS = CG * IDXROW
NCH_AB = B * C // ROWS
ITER_AB = -(-NCH_AB // NW)
NCH_L = B * F // ROWS
ITER_L = -(-NCH_L // NW)

def _sc_gather_body(ffm_hbm, lin_hbm, idxa_hbm, idxb_hbm, idxl_hbm,
               ema_hbm, emb_hbm, linv_hbm,
               idxa_v, idxb_v, idxl_v, rowsa_v, rowsb_v, rowsl_v,
               sema, semb, seml):
    wid = lax.axis_index("s") * NC + lax.axis_index("c")

    def chunk(k, carry):
        m = k * NW + wid

        @pl.when(m < NCH_AB)
        def _():
            pltpu.sync_copy(idxa_hbm.at[m], idxa_v)
            pltpu.sync_copy(idxb_hbm.at[m], idxb_v)
            cps = []
            for g in range(CG):
                cps.append(pltpu.async_copy(
                    ffm_hbm.at[idxa_v.at[g]],
                    rowsa_v.at[pl.ds(g * IDXROW, IDXROW)], sema))
                cps.append(pltpu.async_copy(
                    ffm_hbm.at[idxb_v.at[g]],
                    rowsb_v.at[pl.ds(g * IDXROW, IDXROW)], semb))
            for cp in cps:
                cp.wait()
            pltpu.sync_copy(rowsa_v, ema_hbm.at[pl.ds(m * ROWS, ROWS)])
            pltpu.sync_copy(rowsb_v, emb_hbm.at[pl.ds(m * ROWS, ROWS)])

        return carry

    lax.fori_loop(0, ITER_AB, chunk, 0)

    def lchunk(k, carry):
        m = k * NW + wid

        @pl.when(m < NCH_L)
        def _():
            pltpu.sync_copy(idxl_hbm.at[m], idxl_v)
            cps = [pltpu.async_copy(lin_hbm.at[idxl_v.at[g]],
                                    rowsl_v.at[pl.ds(g * IDXROW, IDXROW)], seml)
                   for g in range(CG)]
            for cp in cps:
                cp.wait()
            pltpu.sync_copy(rowsl_v, linv_hbm.at[pl.ds(m * ROWS, ROWS)])

        return carry

    lax.fori_loop(0, ITER_L, lchunk, 0)


@functools.lru_cache(maxsize=1)
def _sc_gather():
    mesh = plsc.VectorSubcoreMesh(core_axis_name="c", subcore_axis_name="s",
                                  num_cores=NC, num_subcores=NS)
    return pl.kernel(
        _sc_gather_body,
        out_type=(jax.ShapeDtypeStruct((B * C, D), jnp.float32),
                  jax.ShapeDtypeStruct((B * C, D), jnp.float32),
                  jax.ShapeDtypeStruct((B * F, D), jnp.float32)),
        mesh=mesh,
        compiler_params=pltpu.CompilerParams(use_tc_tiling_on_sc=False),
        scratch_types=[
            pltpu.VMEM((CG, IDXROW), jnp.int32),
            pltpu.VMEM((CG, IDXROW), jnp.int32),
            pltpu.VMEM((CG, IDXROW), jnp.int32),
            pltpu.VMEM((ROWS, D), jnp.float32),
            pltpu.VMEM((ROWS, D), jnp.float32),
            pltpu.VMEM((ROWS, D), jnp.float32),
            pltpu.SemaphoreType.DMA,
            pltpu.SemaphoreType.DMA,
            pltpu.SemaphoreType.DMA,
        ],
    )


BB = 128
GRID = B // BB

_G_NP = np.repeat(np.eye(C, dtype=np.float32), D, axis=0)
_GT_NP = np.ascontiguousarray(_G_NP.T)


def _tc_body(ema_ref, emb_ref, linv_ref, cw_ref, cb_ref, g_ref, gt_ref,
             ew1_ref, eb1_ref, ew2_ref, eb2_ref,
             w1_ref, b1_ref, w2_ref, b2_ref, w3_ref, c0_ref, out_ref):
    em = ema_ref[...] * emb_ref[...]
    dcomp = jnp.dot(em * cw_ref[...], g_ref[...],
                    preferred_element_type=jnp.float32) + cb_ref[...]
    t = jnp.maximum(jnp.dot(dcomp, ew1_ref[...],
                            preferred_element_type=jnp.float32) + eb1_ref[...], 0.0)
    s = jnp.maximum(jnp.dot(t, ew2_ref[...],
                            preferred_element_type=jnp.float32) + eb2_ref[...], 0.0)
    aem = em * jnp.dot(s, gt_ref[...], preferred_element_type=jnp.float32)
    h = jnp.maximum(jnp.dot(aem, w1_ref[...],
                            preferred_element_type=jnp.float32) + b1_ref[...], 0.0)
    h = jnp.maximum(jnp.dot(h, w2_ref[...],
                            preferred_element_type=jnp.float32) + b2_ref[...], 0.0)
    ylin = jnp.sum(linv_ref[...], axis=1, keepdims=True)
    y = jnp.sum(h * w3_ref[...], axis=1, keepdims=True) + c0_ref[...] + ylin
    out_ref[...] = 1.0 / (1.0 + jnp.exp(-y))


def _tc_specs():
    zero = lambda i: (0, 0)
    row = lambda i: (i, 0)
    in_specs = [
        pl.BlockSpec((BB, C * D), row),
        pl.BlockSpec((BB, C * D), row),
        pl.BlockSpec((BB, F * D), row),
        pl.BlockSpec((1, C * D), zero),
        pl.BlockSpec((1, C), zero),
        pl.BlockSpec((C * D, C), zero),
        pl.BlockSpec((C, C * D), zero),
        pl.BlockSpec((C, RED), zero),
        pl.BlockSpec((1, RED), zero),
        pl.BlockSpec((RED, C), zero),
        pl.BlockSpec((1, C), zero),
        pl.BlockSpec((C * D, H1), zero),
        pl.BlockSpec((1, H1), zero),
        pl.BlockSpec((H1, H2), zero),
        pl.BlockSpec((1, H2), zero),
        pl.BlockSpec((1, H2), zero),
        pl.BlockSpec((1, 1), zero),
    ]
    out_spec = pl.BlockSpec((BB, 1), row)
    return in_specs, out_spec


def kernel(x, lin_tables, ffm_tables, compose_w, compose_b,
           exc_w1, exc_b1, exc_w2, exc_b2,
           mlp_w1, mlp_b1, mlp_w2, mlp_b2, mlp_w3, mlp_b3, b_global):
    iu = jnp.asarray(_IU, dtype=jnp.int32)
    ju = jnp.asarray(_JU, dtype=jnp.int32)
    ffm_flat = ffm_tables.reshape(F * V * F, D)
    lin_pad = jnp.pad(lin_tables.reshape(F * V, 1), ((0, 0), (0, D - 1)))
    xi = jnp.take(x, iu, axis=1)
    xj = jnp.take(x, ju, axis=1)
    idxa = (iu * (V * F) + ju)[None, :] + xi * F
    idxb = (ju * (V * F) + iu)[None, :] + xj * F
    idxl = (jnp.arange(F, dtype=jnp.int32) * V)[None, :] + x
    idxa2 = idxa.reshape(NCH_AB, CG, IDXROW)
    idxb2 = idxb.reshape(NCH_AB, CG, IDXROW)
    idxl2 = idxl.reshape(NCH_L, CG, IDXROW)

    ema, emb, linv = _sc_gather()(ffm_flat, lin_pad, idxa2, idxb2, idxl2)

    ema2 = ema.reshape(B, C * D)
    emb2 = emb.reshape(B, C * D)
    linv2 = linv.reshape(B, F * D)
    c0 = (mlp_b3[0] + b_global[0]).reshape(1, 1)

    in_specs, out_spec = _tc_specs()
    out = pl.pallas_call(
        _tc_body,
        grid=(GRID,),
        in_specs=in_specs,
        out_specs=out_spec,
        out_shape=jax.ShapeDtypeStruct((B, 1), jnp.float32),
    )(ema2, emb2, linv2,
      compose_w.reshape(1, C * D), compose_b.reshape(1, C),
      jnp.asarray(_G_NP), jnp.asarray(_GT_NP),
      exc_w1, exc_b1.reshape(1, RED), exc_w2, exc_b2.reshape(1, C),
      mlp_w1, mlp_b1.reshape(1, H1), mlp_w2, mlp_b2.reshape(1, H2),
      mlp_w3.reshape(1, H2), c0)
    return out.reshape(B)


# speedup vs baseline: 103.8487x; 1.0047x over previous
import functools

import numpy as np
import jax
import jax.numpy as jnp
from jax import lax
from jax.experimental import pallas as pl
from jax.experimental.pallas import tpu as pltpu
from jax.experimental.pallas import tpu_sc as plsc

B = 4096
F = 26
V = 1000
D = 16
C = F * (F - 1) // 2
RED = C // 2
H1, H2 = 1024, 512

_IU, _JU = np.triu_indices(F, k=1)

NC, NS = 2, 16
NW = NC * NS
IDXROW = 128
CG = 8
ROWS = CG * IDXROW
NCH_AB = B * C // ROWS
ITER_AB = -(-NCH_AB // NW)
NCH_L = B * F // ROWS
ITER_L = -(-NCH_L // NW)

def _sc_gather_body(ffm_hbm, lin_hbm, idxa_hbm, idxb_hbm, idxl_hbm,
               ema_hbm, emb_hbm, linv_hbm,
               idxa_v, idxb_v, idxl_v, rowsa_v, rowsb_v, rowsl_v,
               sema, semb, seml):
    wid = lax.axis_index("s") * NC + lax.axis_index("c")

    def chunk(k, carry):
        m = k * NW + wid

        @pl.when(m < NCH_AB)
        def _():
            pltpu.sync_copy(idxa_hbm.at[m], idxa_v)
            pltpu.sync_copy(idxb_hbm.at[m], idxb_v)
            cps = []
            for g in range(CG):
                cps.append(pltpu.async_copy(
                    ffm_hbm.at[idxa_v.at[g]],
                    rowsa_v.at[pl.ds(g * IDXROW, IDXROW)], sema))
                cps.append(pltpu.async_copy(
                    ffm_hbm.at[idxb_v.at[g]],
                    rowsb_v.at[pl.ds(g * IDXROW, IDXROW)], semb))
            for cp in cps:
                cp.wait()
            pltpu.sync_copy(rowsa_v, ema_hbm.at[pl.ds(m * ROWS, ROWS)])
            pltpu.sync_copy(rowsb_v, emb_hbm.at[pl.ds(m * ROWS, ROWS)])

        return carry

    lax.fori_loop(0, ITER_AB, chunk, 0)

    def lchunk(k, carry):
        m = k * NW + wid

        @pl.when(m < NCH_L)
        def _():
            pltpu.sync_copy(idxl_hbm.at[m], idxl_v)
            cps = [pltpu.async_copy(lin_hbm.at[idxl_v.at[g]],
                                    rowsl_v.at[pl.ds(g * IDXROW, IDXROW)], seml)
                   for g in range(CG)]
            for cp in cps:
                cp.wait()
            pltpu.sync_copy(rowsl_v, linv_hbm.at[pl.ds(m * ROWS, ROWS)])

        return carry

    lax.fori_loop(0, ITER_L, lchunk, 0)


@functools.lru_cache(maxsize=1)
def _sc_gather():
    mesh = plsc.VectorSubcoreMesh(core_axis_name="c", subcore_axis_name="s",
                                  num_cores=NC, num_subcores=NS)
    return pl.kernel(
        _sc_gather_body,
        out_type=(jax.ShapeDtypeStruct((B * C, D), jnp.float32),
                  jax.ShapeDtypeStruct((B * C, D), jnp.float32),
                  jax.ShapeDtypeStruct((B * F, D), jnp.float32)),
        mesh=mesh,
        compiler_params=pltpu.CompilerParams(use_tc_tiling_on_sc=False),
        scratch_types=[
            pltpu.VMEM((CG, IDXROW), jnp.int32),
            pltpu.VMEM((CG, IDXROW), jnp.int32),
            pltpu.VMEM((CG, IDXROW), jnp.int32),
            pltpu.VMEM((ROWS, D), jnp.float32),
            pltpu.VMEM((ROWS, D), jnp.float32),
            pltpu.VMEM((ROWS, D), jnp.float32),
            pltpu.SemaphoreType.DMA,
            pltpu.SemaphoreType.DMA,
            pltpu.SemaphoreType.DMA,
        ],
    )


BB = 128
GRID = B // BB

_G_NP = np.repeat(np.eye(C, dtype=np.float32), D, axis=0)
_GT_NP = np.ascontiguousarray(_G_NP.T)


def _tc_body(ema_ref, emb_ref, linv_ref, cw_ref, cb_ref, g_ref, gt_ref,
             ew1_ref, eb1_ref, ew2_ref, eb2_ref,
             w1_ref, b1_ref, w2_ref, b2_ref, w3_ref, c0_ref, out_ref):
    em = ema_ref[...] * emb_ref[...]
    emw = (em * cw_ref[...]).astype(jnp.bfloat16)
    dcomp = jnp.dot(emw, g_ref[...],
                    preferred_element_type=jnp.float32) + cb_ref[...]
    t = jnp.maximum(jnp.dot(dcomp, ew1_ref[...],
                            preferred_element_type=jnp.float32) + eb1_ref[...], 0.0)
    s = jnp.maximum(jnp.dot(t, ew2_ref[...],
                            preferred_element_type=jnp.float32) + eb2_ref[...], 0.0)
    sexp = jnp.dot(s.astype(jnp.bfloat16), gt_ref[...],
                   preferred_element_type=jnp.float32)
    aem = (em * sexp).astype(jnp.bfloat16)
    h = jnp.maximum(jnp.dot(aem, w1_ref[...],
                            preferred_element_type=jnp.float32) + b1_ref[...], 0.0)
    h = jnp.maximum(jnp.dot(h.astype(jnp.bfloat16), w2_ref[...],
                            preferred_element_type=jnp.float32) + b2_ref[...], 0.0)
    ylin = jnp.sum(linv_ref[...], axis=1, keepdims=True)
    y = jnp.sum(h * w3_ref[...], axis=1, keepdims=True) + c0_ref[...] + ylin
    out_ref[...] = 1.0 / (1.0 + jnp.exp(-y))


def _tc_specs():
    zero = lambda i: (0, 0)
    row = lambda i: (i, 0)
    in_specs = [
        pl.BlockSpec((BB, C * D), row),
        pl.BlockSpec((BB, C * D), row),
        pl.BlockSpec((BB, F * D), row),
        pl.BlockSpec((1, C * D), zero),
        pl.BlockSpec((1, C), zero),
        pl.BlockSpec((C * D, C), zero),
        pl.BlockSpec((C, C * D), zero),
        pl.BlockSpec((C, RED), zero),
        pl.BlockSpec((1, RED), zero),
        pl.BlockSpec((RED, C), zero),
        pl.BlockSpec((1, C), zero),
        pl.BlockSpec((C * D, H1), zero),
        pl.BlockSpec((1, H1), zero),
        pl.BlockSpec((H1, H2), zero),
        pl.BlockSpec((1, H2), zero),
        pl.BlockSpec((1, H2), zero),
        pl.BlockSpec((1, 1), zero),
    ]
    out_spec = pl.BlockSpec((BB, 1), row)
    return in_specs, out_spec


def kernel(x, lin_tables, ffm_tables, compose_w, compose_b,
           exc_w1, exc_b1, exc_w2, exc_b2,
           mlp_w1, mlp_b1, mlp_w2, mlp_b2, mlp_w3, mlp_b3, b_global):
    iu = jnp.asarray(_IU, dtype=jnp.int32)
    ju = jnp.asarray(_JU, dtype=jnp.int32)
    ffm_flat = ffm_tables.reshape(F * V * F, D)
    lin_pad = jnp.pad(lin_tables.reshape(F * V, 1), ((0, 0), (0, D - 1)))
    xi = jnp.take(x, iu, axis=1)
    xj = jnp.take(x, ju, axis=1)
    idxa = (iu * (V * F) + ju)[None, :] + xi * F
    idxb = (ju * (V * F) + iu)[None, :] + xj * F
    idxl = (jnp.arange(F, dtype=jnp.int32) * V)[None, :] + x
    idxa2 = idxa.reshape(NCH_AB, CG, IDXROW)
    idxb2 = idxb.reshape(NCH_AB, CG, IDXROW)
    idxl2 = idxl.reshape(NCH_L, CG, IDXROW)

    ema, emb, linv = _sc_gather()(ffm_flat, lin_pad, idxa2, idxb2, idxl2)

    ema2 = ema.reshape(B, C * D)
    emb2 = emb.reshape(B, C * D)
    linv2 = linv.reshape(B, F * D)
    c0 = (mlp_b3[0] + b_global[0]).reshape(1, 1)

    in_specs, out_spec = _tc_specs()
    out = pl.pallas_call(
        _tc_body,
        grid=(GRID,),
        in_specs=in_specs,
        out_specs=out_spec,
        out_shape=jax.ShapeDtypeStruct((B, 1), jnp.float32),
    )(ema2, emb2, linv2,
      compose_w.reshape(1, C * D), compose_b.reshape(1, C),
      jnp.asarray(_G_NP, dtype=jnp.bfloat16), jnp.asarray(_GT_NP, dtype=jnp.bfloat16),
      exc_w1, exc_b1.reshape(1, RED), exc_w2, exc_b2.reshape(1, C),
      mlp_w1.astype(jnp.bfloat16), mlp_b1.reshape(1, H1),
      mlp_w2.astype(jnp.bfloat16), mlp_b2.reshape(1, H2),
      mlp_w3.reshape(1, H2), c0)
    return out.reshape(B)


# speedup vs baseline: 103.9449x; 1.0009x over previous
import functools

import numpy as np
import jax
import jax.numpy as jnp
from jax import lax
from jax.experimental import pallas as pl
from jax.experimental.pallas import tpu as pltpu
from jax.experimental.pallas import tpu_sc as plsc

B = 4096
F = 26
V = 1000
D = 16
C = F * (F - 1) // 2
RED = C // 2
H1, H2 = 1024, 512

_IU, _JU = np.triu_indices(F, k=1)

NC, NS = 2, 16
NW = NC * NS
IDXROW = 128
CG = 8
ROWS = CG * IDXROW
NCH_AB = B * C // ROWS
ITER_AB = -(-NCH_AB // NW)
NCH_L = B * F // ROWS
ITER_L = -(-NCH_L // NW)

def _sc_gather_body(ffm_hbm, lin_hbm, idxa_hbm, idxb_hbm, idxl_hbm,
               ema_hbm, emb_hbm, linv_hbm,
               idxa_v, idxb_v, idxl_v, rowsa_v, rowsb_v, rowsl_v,
               sema, semb, seml):
    wid = lax.axis_index("s") * NC + lax.axis_index("c")

    def chunk(k, carry):
        m = k * NW + wid

        @pl.when(m < NCH_AB)
        def _():
            pltpu.sync_copy(idxa_hbm.at[m], idxa_v)
            pltpu.sync_copy(idxb_hbm.at[m], idxb_v)
            cps = []
            for g in range(CG):
                cps.append(pltpu.async_copy(
                    ffm_hbm.at[idxa_v.at[g]],
                    rowsa_v.at[pl.ds(g * IDXROW, IDXROW)], sema))
                cps.append(pltpu.async_copy(
                    ffm_hbm.at[idxb_v.at[g]],
                    rowsb_v.at[pl.ds(g * IDXROW, IDXROW)], semb))
            for cp in cps:
                cp.wait()
            pltpu.sync_copy(rowsa_v, ema_hbm.at[pl.ds(m * ROWS, ROWS)])
            pltpu.sync_copy(rowsb_v, emb_hbm.at[pl.ds(m * ROWS, ROWS)])

        return carry

    lax.fori_loop(0, ITER_AB, chunk, 0)

    def lchunk(k, carry):
        m = k * NW + wid

        @pl.when(m < NCH_L)
        def _():
            pltpu.sync_copy(idxl_hbm.at[m], idxl_v)
            cps = [pltpu.async_copy(lin_hbm.at[idxl_v.at[g]],
                                    rowsl_v.at[pl.ds(g * IDXROW, IDXROW)], seml)
                   for g in range(CG)]
            for cp in cps:
                cp.wait()
            pltpu.sync_copy(rowsl_v, linv_hbm.at[pl.ds(m * ROWS, ROWS)])

        return carry

    lax.fori_loop(0, ITER_L, lchunk, 0)


@functools.lru_cache(maxsize=1)
def _sc_gather():
    mesh = plsc.VectorSubcoreMesh(core_axis_name="c", subcore_axis_name="s",
                                  num_cores=NC, num_subcores=NS)
    return pl.kernel(
        _sc_gather_body,
        out_type=(jax.ShapeDtypeStruct((B * C, D), jnp.float32),
                  jax.ShapeDtypeStruct((B * C, D), jnp.float32),
                  jax.ShapeDtypeStruct((B * F, D), jnp.float32)),
        mesh=mesh,
        compiler_params=pltpu.CompilerParams(use_tc_tiling_on_sc=False),
        scratch_types=[
            pltpu.VMEM((CG, IDXROW), jnp.int32),
            pltpu.VMEM((CG, IDXROW), jnp.int32),
            pltpu.VMEM((CG, IDXROW), jnp.int32),
            pltpu.VMEM((ROWS, D), jnp.float32),
            pltpu.VMEM((ROWS, D), jnp.float32),
            pltpu.VMEM((ROWS, D), jnp.float32),
            pltpu.SemaphoreType.DMA,
            pltpu.SemaphoreType.DMA,
            pltpu.SemaphoreType.DMA,
        ],
    )


BB = 128
GRID = B // BB

_G_NP = np.repeat(np.eye(C, dtype=np.float32), D, axis=0)
_GT_NP = np.ascontiguousarray(_G_NP.T)


def _tc_body(ema_ref, emb_ref, linv_ref, cw_ref, cb_ref, g_ref, gt_ref,
             ew1_ref, eb1_ref, ew2_ref, eb2_ref,
             w1_ref, b1_ref, w2_ref, b2_ref, w3_ref, c0_ref, out_ref):
    em = ema_ref[...] * emb_ref[...]
    emw = (em * cw_ref[...]).astype(jnp.bfloat16)
    dcomp = jnp.dot(emw, g_ref[...],
                    preferred_element_type=jnp.float32) + cb_ref[...]
    t = jnp.maximum(jnp.dot(dcomp, ew1_ref[...],
                            preferred_element_type=jnp.float32) + eb1_ref[...], 0.0)
    s = jnp.maximum(jnp.dot(t, ew2_ref[...],
                            preferred_element_type=jnp.float32) + eb2_ref[...], 0.0)
    sexp = jnp.dot(s.astype(jnp.bfloat16), gt_ref[...],
                   preferred_element_type=jnp.float32)
    aem = (em * sexp).astype(jnp.bfloat16)
    h = jnp.maximum(jnp.dot(aem, w1_ref[...],
                            preferred_element_type=jnp.float32) + b1_ref[...], 0.0)
    h = jnp.maximum(jnp.dot(h.astype(jnp.bfloat16), w2_ref[...],
                            preferred_element_type=jnp.float32) + b2_ref[...], 0.0)
    ylin = jnp.sum(linv_ref[...], axis=1, keepdims=True)
    y = jnp.sum(h * w3_ref[...], axis=1, keepdims=True) + c0_ref[...] + ylin
    out_ref[...] = 1.0 / (1.0 + jnp.exp(-y))


def _tc_specs():
    zero = lambda i: (0, 0)
    row = lambda i: (i, 0)
    in_specs = [
        pl.BlockSpec((BB, C * D), row),
        pl.BlockSpec((BB, C * D), row),
        pl.BlockSpec((BB, F * D), row),
        pl.BlockSpec((1, C * D), zero),
        pl.BlockSpec((1, C), zero),
        pl.BlockSpec((C * D, C), zero),
        pl.BlockSpec((C, C * D), zero),
        pl.BlockSpec((C, RED), zero),
        pl.BlockSpec((1, RED), zero),
        pl.BlockSpec((RED, C), zero),
        pl.BlockSpec((1, C), zero),
        pl.BlockSpec((C * D, H1), zero),
        pl.BlockSpec((1, H1), zero),
        pl.BlockSpec((H1, H2), zero),
        pl.BlockSpec((1, H2), zero),
        pl.BlockSpec((1, H2), zero),
        pl.BlockSpec((1, 1), zero),
    ]
    out_spec = pl.BlockSpec((BB, 1), row)
    return in_specs, out_spec


def kernel(x, lin_tables, ffm_tables, compose_w, compose_b,
           exc_w1, exc_b1, exc_w2, exc_b2,
           mlp_w1, mlp_b1, mlp_w2, mlp_b2, mlp_w3, mlp_b3, b_global):
    iu = jnp.asarray(_IU, dtype=jnp.int32)
    ju = jnp.asarray(_JU, dtype=jnp.int32)
    ffm_g = jax.lax.optimization_barrier(
        ffm_tables.reshape(F, V * F // 8, 8, D).reshape(F * V * F // 8, 8 * D))
    ffm_flat = ffm_g.reshape(F * V * F, D)
    lin_pad = jnp.pad(lin_tables.reshape(F * V, 1), ((0, 0), (0, D - 1)))
    xi = jnp.take(x, iu, axis=1)
    xj = jnp.take(x, ju, axis=1)
    idxa = (iu * (V * F) + ju)[None, :] + xi * F
    idxb = (ju * (V * F) + iu)[None, :] + xj * F
    idxl = (jnp.arange(F, dtype=jnp.int32) * V)[None, :] + x
    idxa2 = idxa.reshape(NCH_AB, CG, IDXROW)
    idxb2 = idxb.reshape(NCH_AB, CG, IDXROW)
    idxl2 = idxl.reshape(NCH_L, CG, IDXROW)

    ema, emb, linv = _sc_gather()(ffm_flat, lin_pad, idxa2, idxb2, idxl2)

    ema2 = ema.reshape(B, C * D)
    emb2 = emb.reshape(B, C * D)
    linv2 = linv.reshape(B, F * D)
    c0 = (mlp_b3[0] + b_global[0]).reshape(1, 1)

    in_specs, out_spec = _tc_specs()
    out = pl.pallas_call(
        _tc_body,
        grid=(GRID,),
        in_specs=in_specs,
        out_specs=out_spec,
        out_shape=jax.ShapeDtypeStruct((B, 1), jnp.float32),
    )(ema2, emb2, linv2,
      compose_w.reshape(1, C * D), compose_b.reshape(1, C),
      jnp.asarray(_G_NP, dtype=jnp.bfloat16), jnp.asarray(_GT_NP, dtype=jnp.bfloat16),
      exc_w1, exc_b1.reshape(1, RED), exc_w2, exc_b2.reshape(1, C),
      mlp_w1.astype(jnp.bfloat16), mlp_b1.reshape(1, H1),
      mlp_w2.astype(jnp.bfloat16), mlp_b2.reshape(1, H2),
      mlp_w3.reshape(1, H2), c0)
    return out.reshape(B)
